# Initial kernel scaffold; baseline (speedup 1.0000x reference)
import jax
import jax.numpy as jnp
from jax.experimental import pallas as pl


def kernel(x, edge_index, W1, b1, W2, b2):
    raise NotImplementedError("write your pallas kernel here")


# speedup vs baseline: 31.3225x; 31.3225x over previous
import functools

import jax
import jax.numpy as jnp
from jax import lax
from jax.experimental import pallas as pl
from jax.experimental.pallas import tpu as pltpu
from jax.experimental.pallas import tpu_sc as plsc

L = 16
NC = 2
NS = 16
NW = NC * NS
CH = 128


def _mesh():
    return plsc.VectorSubcoreMesh(
        core_axis_name="c", subcore_axis_name="s", num_cores=NC, num_subcores=NS)


_SC_PARAMS = pltpu.CompilerParams(use_tc_tiling_on_sc=False)


def _make_deg(n, nchunk):
    rpt = n // NS

    @functools.partial(
        pl.kernel,
        mesh=_mesh(),
        out_type=jax.ShapeDtypeStruct((NC, n, L), jnp.float32),
        scratch_types=[
            pltpu.VMEM((nchunk, CH), jnp.int32),
            pltpu.VMEM((rpt, L), jnp.float32),
            pltpu.VMEM_SHARED((n + L, L), jnp.float32),
        ],
        compiler_params=_SC_PARAMS,
    )
    def deg_kernel(dst_hbm, out_hbm, idx_v, ones_v, acc_sh):
        c = lax.axis_index("c")
        s = lax.axis_index("s")
        w = s * NC + c

        def fill(i, _):
            ones_v[i] = jnp.full((L,), 1.0, jnp.float32)
            return 0

        lax.fori_loop(0, rpt, fill, 0)
        pltpu.sync_copy(dst_hbm.at[w], idx_v)
        pltpu.sync_copy(ones_v, acc_sh.at[pl.ds(s * rpt, rpt)])
        plsc.subcore_barrier()

        def body(j, _):
            pltpu.sync_copy(ones_v.at[pl.ds(0, CH)], acc_sh.at[idx_v.at[j]],
                            add=True)
            return 0

        lax.fori_loop(0, nchunk, body, 0)
        plsc.subcore_barrier()
        pltpu.sync_copy(acc_sh.at[pl.ds(s * rpt, rpt)],
                        out_hbm.at[c, pl.ds(s * rpt, rpt)])

    return deg_kernel


def _make_agg(n, nchunk):
    rpt = n // NS

    @functools.partial(
        pl.kernel,
        mesh=_mesh(),
        out_type=jax.ShapeDtypeStruct((NC, n, L), jnp.float32),
        scratch_types=[
            pltpu.VMEM((nchunk, CH), jnp.int32),
            pltpu.VMEM((nchunk, CH), jnp.int32),
            pltpu.VMEM((CH, L), jnp.float32),
            pltpu.VMEM((rpt, L), jnp.float32),
            pltpu.VMEM_SHARED((n + L, L), jnp.float32),
        ],
        compiler_params=_SC_PARAMS,
    )
    def agg_kernel(src_hbm, dst_hbm, hp_hbm, out_hbm,
                   sidx_v, didx_v, rows_v, init_v, acc_sh):
        c = lax.axis_index("c")
        s = lax.axis_index("s")
        w = s * NC + c

        pltpu.sync_copy(src_hbm.at[w], sidx_v)
        pltpu.sync_copy(dst_hbm.at[w], didx_v)
        pltpu.sync_copy(hp_hbm.at[pl.ds(s * rpt, rpt)], init_v)
        pltpu.sync_copy(init_v, acc_sh.at[pl.ds(s * rpt, rpt)])
        plsc.subcore_barrier()

        def body(j, _):
            pltpu.sync_copy(hp_hbm.at[sidx_v.at[j]], rows_v)
            pltpu.sync_copy(rows_v, acc_sh.at[didx_v.at[j]], add=True)
            return 0

        lax.fori_loop(0, nchunk, body, 0)
        plsc.subcore_barrier()
        pltpu.sync_copy(acc_sh.at[pl.ds(s * rpt, rpt)],
                        out_hbm.at[c, pl.ds(s * rpt, rpt)])

    return agg_kernel


def _dense1_body(x_ref, w1_ref, dega_ref, dinv_ref, hp_ref):
    deg = dega_ref[0] + dega_ref[1] - 1.0
    dinv = lax.rsqrt(deg)
    h = jnp.dot(x_ref[...], w1_ref[...], preferred_element_type=jnp.float32)
    dinv_ref[...] = dinv
    hp_ref[...] = h * dinv


def _dense2_body(acc_ref, hp_ref, dinv_ref, b1_ref, w2_ref, hp2_ref):
    agg = acc_ref[0] + acc_ref[1] - hp_ref[...]
    out1 = jnp.maximum(dinv_ref[...] * agg + b1_ref[...], 0.0)
    hp2_ref[...] = jnp.dot(out1, w2_ref[...],
                           preferred_element_type=jnp.float32) * dinv_ref[...]


def _dense3_body(c, acc_ref, hp2_ref, dinv_ref, b2_ref, out_ref):
    t = dinv_ref[...] * (acc_ref[0] + acc_ref[1] - hp2_ref[...])
    logits = t[:, :c] + b2_ref[...]
    m = jnp.max(logits, axis=1, keepdims=True)
    lse = jnp.log(jnp.sum(jnp.exp(logits - m), axis=1, keepdims=True)) + m
    out_ref[...] = logits - lse


def kernel(x, edge_index, W1, b1, W2, b2):
    n, _ = x.shape
    h = W1.shape[1]
    c = W2.shape[1]
    e = edge_index.shape[1]
    assert h == L and c <= L and n % NS == 0

    nchunk = -(-e // (NW * CH))
    ep = nchunk * CH * NW
    src = jnp.concatenate(
        [edge_index[0], jnp.zeros((ep - e,), jnp.int32)]).reshape(NW, nchunk, CH)
    dst = jnp.concatenate(
        [edge_index[1], jnp.full((ep - e,), n, jnp.int32)]).reshape(NW, nchunk, CH)

    deg_call = _make_deg(n, nchunk)
    agg_call = _make_agg(n, nchunk)

    dega = deg_call(dst)

    f32 = jnp.float32
    dinvb, hp = pl.pallas_call(
        _dense1_body,
        out_shape=(jax.ShapeDtypeStruct((n, L), f32),
                   jax.ShapeDtypeStruct((n, L), f32)),
    )(x, W1, dega)

    acc1 = agg_call(src, dst, hp)

    w2p = jnp.pad(W2, ((0, 0), (0, L - c)))
    hp2 = pl.pallas_call(
        _dense2_body,
        out_shape=jax.ShapeDtypeStruct((n, L), f32),
    )(acc1, hp, dinvb, b1.reshape(1, h), w2p)

    acc2 = agg_call(src, dst, hp2)

    out = pl.pallas_call(
        functools.partial(_dense3_body, c),
        out_shape=jax.ShapeDtypeStruct((n, c), f32),
    )(acc2, hp2, dinvb, b2.reshape(1, c))
    return out


# speedup vs baseline: 79.4407x; 2.5362x over previous
import functools

import jax
import jax.numpy as jnp
from jax import lax
from jax.experimental import pallas as pl
from jax.experimental.pallas import tpu as pltpu
from jax.experimental.pallas import tpu_sc as plsc

L = 16
NC = 2
NS = 16
NW = NC * NS
CH = 128
K = 8


def _mesh():
    return plsc.VectorSubcoreMesh(
        core_axis_name="c", subcore_axis_name="s", num_cores=NC, num_subcores=NS)


_SC_PARAMS = pltpu.CompilerParams(use_tc_tiling_on_sc=False)


def _make_deg(n, nchunk):
    rpt = n // NS

    @functools.partial(
        pl.kernel,
        mesh=_mesh(),
        out_type=jax.ShapeDtypeStruct((NC, n, L), jnp.float32),
        scratch_types=[
            pltpu.VMEM((nchunk, CH), jnp.int32),
            pltpu.VMEM((rpt, L), jnp.float32),
            pltpu.VMEM_SHARED((n + CH, L), jnp.float32),
            pltpu.SemaphoreType.DMA,
        ],
        compiler_params=_SC_PARAMS,
    )
    def deg_kernel(dst_hbm, out_hbm, idx_v, ones_v, acc_sh, sem):
        c = lax.axis_index("c")
        s = lax.axis_index("s")
        w = s * NC + c

        def fill(i, _):
            ones_v[i] = jnp.full((L,), 1.0, jnp.float32)
            return 0

        lax.fori_loop(0, rpt, fill, 0)
        pltpu.sync_copy(dst_hbm.at[w], idx_v)
        pltpu.sync_copy(ones_v, acc_sh.at[pl.ds(s * rpt, rpt)])
        plsc.subcore_barrier()

        def fire(j, _):
            pltpu.async_copy(ones_v.at[pl.ds(0, CH)], acc_sh.at[idx_v.at[j]],
                             sem, add=True)
            return 0

        lax.fori_loop(0, nchunk, fire, 0)

        def drain(j, _):
            pltpu.make_async_copy(ones_v.at[pl.ds(0, CH)],
                                  acc_sh.at[idx_v.at[j]], sem).wait()
            return 0

        lax.fori_loop(0, nchunk, drain, 0)
        plsc.subcore_barrier()
        pltpu.sync_copy(acc_sh.at[pl.ds(s * rpt, rpt)],
                        out_hbm.at[c, pl.ds(s * rpt, rpt)])

    return deg_kernel


def _make_agg(n, nchunk):
    rpt = n // NS
    ngroups = nchunk // K
    assert nchunk % K == 0 and ngroups % 2 == 0

    @functools.partial(
        pl.kernel,
        mesh=_mesh(),
        out_type=jax.ShapeDtypeStruct((NC, n, L), jnp.float32),
        scratch_types=[
            pltpu.VMEM((nchunk, CH), jnp.int32),
            pltpu.VMEM((nchunk, CH), jnp.int32),
            pltpu.VMEM((2, K, CH, L), jnp.float32),
            pltpu.VMEM((rpt, L), jnp.float32),
            pltpu.VMEM_SHARED((n + CH, L), jnp.float32),
            pltpu.SemaphoreType.DMA,
            pltpu.SemaphoreType.DMA,
            pltpu.SemaphoreType.DMA,
            pltpu.SemaphoreType.DMA,
        ],
        compiler_params=_SC_PARAMS,
    )
    def agg_kernel(src_hbm, dst_hbm, hp_hbm, out_hbm,
                   sidx_v, didx_v, rows_v, init_v, acc_sh,
                   gsem0, gsem1, ssem0, ssem1):
        c = lax.axis_index("c")
        s = lax.axis_index("s")
        w = s * NC + c
        gsem = (gsem0, gsem1)
        ssem = (ssem0, ssem1)

        pltpu.sync_copy(src_hbm.at[w], sidx_v)
        pltpu.sync_copy(dst_hbm.at[w], didx_v)
        pltpu.sync_copy(hp_hbm.at[pl.ds(s * rpt, rpt)], init_v)
        pltpu.sync_copy(init_v, acc_sh.at[pl.ds(s * rpt, rpt)])
        plsc.subcore_barrier()

        for b in range(K):
            pltpu.async_copy(hp_hbm.at[sidx_v.at[b]], rows_v.at[0, b], gsem[0])

        def half(g, p):
            q = 1 - p

            @pl.when(g >= 1)
            def _():
                for b in range(K):
                    pltpu.make_async_copy(
                        rows_v.at[q, b], acc_sh.at[didx_v.at[(g - 1) * K + b]],
                        ssem[q]).wait()

            @pl.when(g + 1 < ngroups)
            def _():
                for b in range(K):
                    pltpu.async_copy(hp_hbm.at[sidx_v.at[(g + 1) * K + b]],
                                     rows_v.at[q, b], gsem[q])

            for b in range(K):
                pltpu.make_async_copy(hp_hbm.at[sidx_v.at[g * K + b]],
                                      rows_v.at[p, b], gsem[p]).wait()
            for b in range(K):
                pltpu.async_copy(rows_v.at[p, b],
                                 acc_sh.at[didx_v.at[g * K + b]],
                                 ssem[p], add=True)

        def pair(g2, _):
            half(2 * g2, 0)
            half(2 * g2 + 1, 1)
            return 0

        lax.fori_loop(0, ngroups // 2, pair, 0)
        for b in range(K):
            pltpu.make_async_copy(
                rows_v.at[1, b], acc_sh.at[didx_v.at[(ngroups - 1) * K + b]],
                ssem[1]).wait()
        plsc.subcore_barrier()
        pltpu.sync_copy(acc_sh.at[pl.ds(s * rpt, rpt)],
                        out_hbm.at[c, pl.ds(s * rpt, rpt)])

    return agg_kernel


def _dense1_body(xp_ref, w1s_ref, dega_ref, dinv_ref, hp_ref):
    deg = dega_ref[0] + dega_ref[1] - 1.0
    dinv = lax.rsqrt(deg)
    h = jnp.dot(xp_ref[...], w1s_ref[...], preferred_element_type=jnp.float32)
    dinv_ref[...] = dinv
    hp_ref[...] = h * dinv


def _dense2_body(acc_ref, hp_ref, dinv_ref, b1_ref, w2bd_ref, hp2_ref):
    agg = acc_ref[0] + acc_ref[1] - hp_ref[...]
    out1 = jnp.maximum(dinv_ref[...] * agg + b1_ref[...], 0.0)
    hp2_ref[...] = jnp.dot(out1, w2bd_ref[...],
                           preferred_element_type=jnp.float32) * dinv_ref[...]


def _dense3_body(c, acc_ref, hp2_ref, dinv_ref, b2_ref, out_ref):
    t = dinv_ref[...] * (acc_ref[0] + acc_ref[1] - hp2_ref[...])
    t3 = t.reshape(t.shape[0], 8, L)[:, :, :c]
    logits = t3 + b2_ref[...].reshape(1, 1, c)
    m = jnp.max(logits, axis=2, keepdims=True)
    lse = jnp.log(jnp.sum(jnp.exp(logits - m), axis=2, keepdims=True)) + m
    out_ref[...] = (logits - lse).reshape(t.shape[0], 8 * c)


def kernel(x, edge_index, W1, b1, W2, b2):
    n, d = x.shape
    h = W1.shape[1]
    c = W2.shape[1]
    e = edge_index.shape[1]
    assert h == L and n % NS == 0 and n % 8 == 0

    nchunk = -(-e // (NW * CH * K)) * K
    ep = nchunk * CH * NW
    pad_ar = jnp.arange(ep - e, dtype=jnp.int32)
    src = jnp.concatenate(
        [edge_index[0], pad_ar % n]).reshape(NW, nchunk, CH)
    dst = jnp.concatenate(
        [edge_index[1], n + pad_ar % CH]).reshape(NW, nchunk, CH)

    deg_call = _make_deg(n, nchunk)
    agg_call = _make_agg(n, nchunk)

    dega = deg_call(dst)

    rp = n // 8
    f32 = jnp.float32
    eye8 = jnp.eye(8, dtype=f32)
    w1s = (eye8[:, None, :, None] * W1[None, :, None, :]).reshape(8 * d, 128)
    w2p = jnp.pad(W2, ((0, 0), (0, L - c)))
    w2bd = (eye8[:, None, :, None] * w2p[None, :, None, :]).reshape(128, 128)
    xp = x.reshape(rp, 8 * d)
    b1t = jnp.tile(b1, 8).reshape(1, 128)

    dinvp, hpp = pl.pallas_call(
        _dense1_body,
        out_shape=(jax.ShapeDtypeStruct((rp, 128), f32),
                   jax.ShapeDtypeStruct((rp, 128), f32)),
    )(xp, w1s, dega.reshape(NC, rp, 128))

    acc1 = agg_call(src, dst, hpp.reshape(n, L))

    hp2p = pl.pallas_call(
        _dense2_body,
        out_shape=jax.ShapeDtypeStruct((rp, 128), f32),
    )(acc1.reshape(NC, rp, 128), hpp, dinvp, b1t, w2bd)

    acc2 = agg_call(src, dst, hp2p.reshape(n, L))

    outp = pl.pallas_call(
        functools.partial(_dense3_body, c),
        out_shape=jax.ShapeDtypeStruct((rp, 8 * c), f32),
    )(acc2.reshape(NC, rp, 128), hp2p, dinvp, b2.reshape(1, c))
    return outp.reshape(n, c)


# speedup vs baseline: 79.4584x; 1.0002x over previous
import functools

import jax
import jax.numpy as jnp
from jax import lax
from jax.experimental import pallas as pl
from jax.experimental.pallas import tpu as pltpu
from jax.experimental.pallas import tpu_sc as plsc

L = 16
NC = 2
NS = 16
NW = NC * NS
CH = 128
K = 8


def _mesh():
    return plsc.VectorSubcoreMesh(
        core_axis_name="c", subcore_axis_name="s", num_cores=NC, num_subcores=NS)


_SC_PARAMS = pltpu.CompilerParams(use_tc_tiling_on_sc=False)


def _make_deg(n, nchunk):
    rpt = n // NS

    @functools.partial(
        pl.kernel,
        mesh=_mesh(),
        out_type=jax.ShapeDtypeStruct((NC, n, L), jnp.float32),
        scratch_types=[
            pltpu.VMEM((nchunk, CH), jnp.int32),
            pltpu.VMEM((rpt, L), jnp.float32),
            pltpu.VMEM_SHARED((n + CH, L), jnp.float32),
            pltpu.SemaphoreType.DMA,
        ],
        compiler_params=_SC_PARAMS,
    )
    def deg_kernel(dst_hbm, out_hbm, idx_v, ones_v, acc_sh, sem):
        c = lax.axis_index("c")
        s = lax.axis_index("s")
        w = s * NC + c

        def fill(i, _):
            ones_v[i] = jnp.full((L,), 1.0, jnp.float32)
            return 0

        lax.fori_loop(0, rpt, fill, 0)
        pltpu.sync_copy(dst_hbm.at[w], idx_v)
        pltpu.sync_copy(ones_v, acc_sh.at[pl.ds(s * rpt, rpt)])
        plsc.subcore_barrier()

        def fire(j, _):
            pltpu.async_copy(ones_v.at[pl.ds(0, CH)], acc_sh.at[idx_v.at[j]],
                             sem, add=True)
            return 0

        lax.fori_loop(0, nchunk, fire, 0)

        def drain(j, _):
            pltpu.make_async_copy(ones_v.at[pl.ds(0, CH)],
                                  acc_sh.at[idx_v.at[j]], sem).wait()
            return 0

        lax.fori_loop(0, nchunk, drain, 0)
        plsc.subcore_barrier()
        pltpu.sync_copy(acc_sh.at[pl.ds(s * rpt, rpt)],
                        out_hbm.at[c, pl.ds(s * rpt, rpt)])

    return deg_kernel


def _make_agg(n, nchunk):
    rpt = n // NS
    ngroups = nchunk // K
    assert nchunk % K == 0 and ngroups % 2 == 0

    @functools.partial(
        pl.kernel,
        mesh=_mesh(),
        out_type=jax.ShapeDtypeStruct((NC, n, L), jnp.float32),
        scratch_types=[
            pltpu.VMEM((nchunk, CH), jnp.int32),
            pltpu.VMEM((nchunk, CH), jnp.int32),
            pltpu.VMEM((2, K, CH, L), jnp.float32),
            pltpu.VMEM((rpt, L), jnp.float32),
            pltpu.VMEM_SHARED((n + CH, L), jnp.float32),
            pltpu.SemaphoreType.DMA,
            pltpu.SemaphoreType.DMA,
            pltpu.SemaphoreType.DMA,
            pltpu.SemaphoreType.DMA,
        ],
        compiler_params=_SC_PARAMS,
    )
    def agg_kernel(src_hbm, dst_hbm, hp_hbm, out_hbm,
                   sidx_v, didx_v, rows_v, init_v, acc_sh,
                   gsem0, gsem1, ssem0, ssem1):
        c = lax.axis_index("c")
        s = lax.axis_index("s")
        w = s * NC + c
        gsem = (gsem0, gsem1)
        ssem = (ssem0, ssem1)

        pltpu.sync_copy(src_hbm.at[w], sidx_v)
        pltpu.sync_copy(dst_hbm.at[w], didx_v)
        pltpu.sync_copy(hp_hbm.at[pl.ds(s * rpt, rpt)], init_v)
        pltpu.sync_copy(init_v, acc_sh.at[pl.ds(s * rpt, rpt)])
        plsc.subcore_barrier()

        for b in range(K):
            pltpu.async_copy(hp_hbm.at[sidx_v.at[b]], rows_v.at[0, b], gsem[0])

        def half(g, p):
            q = 1 - p

            @pl.when(g >= 1)
            def _():
                for b in range(K):
                    pltpu.make_async_copy(
                        rows_v.at[q, b], acc_sh.at[didx_v.at[(g - 1) * K + b]],
                        ssem[q]).wait()

            @pl.when(g + 1 < ngroups)
            def _():
                for b in range(K):
                    pltpu.async_copy(hp_hbm.at[sidx_v.at[(g + 1) * K + b]],
                                     rows_v.at[q, b], gsem[q])

            for b in range(K):
                pltpu.make_async_copy(hp_hbm.at[sidx_v.at[g * K + b]],
                                      rows_v.at[p, b], gsem[p]).wait()
            for b in range(K):
                pltpu.async_copy(rows_v.at[p, b],
                                 acc_sh.at[didx_v.at[g * K + b]],
                                 ssem[p], add=True)

        def pair(g2, _):
            half(2 * g2, 0)
            half(2 * g2 + 1, 1)
            return 0

        lax.fori_loop(0, ngroups // 2, pair, 0)
        for b in range(K):
            pltpu.make_async_copy(
                rows_v.at[1, b], acc_sh.at[didx_v.at[(ngroups - 1) * K + b]],
                ssem[1]).wait()
        plsc.subcore_barrier()
        pltpu.sync_copy(acc_sh.at[pl.ds(s * rpt, rpt)],
                        out_hbm.at[c, pl.ds(s * rpt, rpt)])

    return agg_kernel


def _dense1_body(xp_ref, w1s_ref, dega_ref, dinv_ref, hp_ref):
    deg = dega_ref[0] + dega_ref[1] - 1.0
    dinv = lax.rsqrt(deg)
    h = jnp.dot(xp_ref[...], w1s_ref[...], preferred_element_type=jnp.float32)
    dinv_ref[...] = dinv
    hp_ref[...] = h * dinv


def _dense2_body(acc_ref, hp_ref, dinv_ref, b1_ref, w2bd_ref, hp2_ref):
    agg = acc_ref[0] + acc_ref[1] - hp_ref[...]
    out1 = jnp.maximum(dinv_ref[...] * agg + b1_ref[...], 0.0)
    hp2_ref[...] = jnp.dot(out1, w2bd_ref[...],
                           preferred_element_type=jnp.float32) * dinv_ref[...]


def _dense3_body(c, acc_ref, hp2_ref, dinv_ref, b2_ref, out_ref):
    t = dinv_ref[...] * (acc_ref[0] + acc_ref[1] - hp2_ref[...])
    t3 = t.reshape(t.shape[0], 8, L)[:, :, :c]
    logits = t3 + b2_ref[...].reshape(1, 1, c)
    m = jnp.max(logits, axis=2, keepdims=True)
    lse = jnp.log(jnp.sum(jnp.exp(logits - m), axis=2, keepdims=True)) + m
    out_ref[...] = (logits - lse).reshape(t.shape[0], 8 * c)


def kernel(x, edge_index, W1, b1, W2, b2):
    n, d = x.shape
    h = W1.shape[1]
    c = W2.shape[1]
    e = edge_index.shape[1]
    assert h == L and n % NS == 0 and n % 8 == 0

    nchunk = -(-e // (NW * CH * K)) * K
    ep = nchunk * CH * NW
    pad_ar = jnp.arange(ep - e, dtype=jnp.int32)
    src_pad = jnp.minimum(pad_ar & 8191, n - 1)
    src = jnp.concatenate(
        [edge_index[0], src_pad]).reshape(NW, nchunk, CH)
    dst = jnp.concatenate(
        [edge_index[1], n + (pad_ar & (CH - 1))]).reshape(NW, nchunk, CH)

    deg_call = _make_deg(n, nchunk)
    agg_call = _make_agg(n, nchunk)

    dega = deg_call(dst)

    rp = n // 8
    f32 = jnp.float32
    eye8 = jnp.eye(8, dtype=f32)
    w1s = (eye8[:, None, :, None] * W1[None, :, None, :]).reshape(8 * d, 128)
    w2p = jnp.pad(W2, ((0, 0), (0, L - c)))
    w2bd = (eye8[:, None, :, None] * w2p[None, :, None, :]).reshape(128, 128)
    xp = x.reshape(rp, 8 * d)
    b1t = jnp.tile(b1, 8).reshape(1, 128)

    dinvp, hpp = pl.pallas_call(
        _dense1_body,
        out_shape=(jax.ShapeDtypeStruct((rp, 128), f32),
                   jax.ShapeDtypeStruct((rp, 128), f32)),
    )(xp, w1s, dega.reshape(NC, rp, 128))

    acc1 = agg_call(src, dst, hpp.reshape(n, L))

    hp2p = pl.pallas_call(
        _dense2_body,
        out_shape=jax.ShapeDtypeStruct((rp, 128), f32),
    )(acc1.reshape(NC, rp, 128), hpp, dinvp, b1t, w2bd)

    acc2 = agg_call(src, dst, hp2p.reshape(n, L))

    outp = pl.pallas_call(
        functools.partial(_dense3_body, c),
        out_shape=jax.ShapeDtypeStruct((rp, 8 * c), f32),
    )(acc2.reshape(NC, rp, 128), hp2p, dinvp, b2.reshape(1, c))
    return outp.reshape(n, c)
